# baseline (speedup 1.0000x reference)
import jax
import jax.numpy as jnp
from jax import lax
from jax.experimental import pallas as pl
from jax.experimental.pallas import tpu as pltpu
from jax.experimental.pallas import tpu_sc as plsc

_SMOOTH = 100.0
_N = 64 * 512 * 512
_COLS = 512
_ROWS = _N // _COLS

_NC = 2
_NS = 16
_NW = _NC * _NS
_CH = 16384
_CPW = 4
_PW = _CPW * _CH
_SC_ROWS = _NW * _PW // _COLS
_L = 16
_U = 8

_BR = 2048
_TC_OFF = _SC_ROWS // _BR
_G = (_ROWS - _SC_ROWS) // _BR


def _sc_body(yt_hbm, yp_hbm, out_hbm, yt_buf, yp_buf, res_buf,
             sem_t0, sem_t1, sem_p0, sem_p1):
    wid = lax.axis_index("s") * _NC + lax.axis_index("c")
    base = wid * _PW
    sem_t = (sem_t0, sem_t1)
    sem_p = (sem_p0, sem_p1)

    def start(k, b):
        pltpu.async_copy(yt_hbm.at[pl.ds(base + k * _CH, _CH)],
                         yt_buf.at[b], sem_t[b])
        pltpu.async_copy(yp_hbm.at[pl.ds(base + k * _CH, _CH)],
                         yp_buf.at[b], sem_p[b])

    start(0, 0)
    start(1, 1)

    zeros = jnp.zeros((_L,), jnp.float32)
    init = (zeros, zeros, zeros, zeros)

    def outer(g, accs):
        for b in range(2):
            k = 2 * g + b
            pltpu.make_async_copy(yt_hbm.at[pl.ds(base, _CH)],
                                  yt_buf.at[b], sem_t[b]).wait()
            pltpu.make_async_copy(yp_hbm.at[pl.ds(base, _CH)],
                                  yp_buf.at[b], sem_p[b]).wait()

            def vec_body(i, accs, b=b):
                accs = list(accs)
                for u in range(_U):
                    off = i * (_U * _L) + u * _L
                    yt = yt_buf[b, pl.ds(off, _L)]
                    a = jnp.abs(yt)
                    p = jnp.abs(yp_buf[b, pl.ds(off, _L)])
                    m = yt != jnp.float32(0.85)
                    a = jnp.where(m, a, jnp.float32(0.0))
                    p = jnp.where(m, p, jnp.float32(0.0))
                    j = u % 2
                    accs[j] = accs[j] + a * p
                    accs[2 + j] = accs[2 + j] + (a + p)
                return tuple(accs)

            accs = lax.fori_loop(0, _CH // (_U * _L), vec_body, accs)

            @pl.when(k + 2 < _CPW)
            def _(k=k, b=b):
                start(k + 2, b)
        return accs

    accs = lax.fori_loop(0, _CPW // 2, outer, init)
    res_buf[pl.ds(0, _L)] = accs[0] + accs[1]
    res_buf[pl.ds(_L, _L)] = accs[2] + accs[3]
    pltpu.sync_copy(res_buf, out_hbm.at[wid])


def _sc_partials(yt_flat, yp_flat):
    return pl.kernel(
        _sc_body,
        out_type=jax.ShapeDtypeStruct((_NW, 2 * _L), jnp.float32),
        mesh=plsc.VectorSubcoreMesh(core_axis_name="c", subcore_axis_name="s"),
        scratch_types=[
            pltpu.VMEM((2, _CH), jnp.float32),
            pltpu.VMEM((2, _CH), jnp.float32),
            pltpu.VMEM((2 * _L,), jnp.float32),
            pltpu.SemaphoreType.DMA,
            pltpu.SemaphoreType.DMA,
            pltpu.SemaphoreType.DMA,
            pltpu.SemaphoreType.DMA,
        ],
    )(yt_flat, yp_flat)


def _tc_body(yt_ref, yp_ref, oi_ref, os_ref):
    pi = [jnp.zeros((8, 128), jnp.float32) for _ in range(4)]
    si = [jnp.zeros((8, 128), jnp.float32) for _ in range(4)]
    for k in range(_BR // 8):
        x = yt_ref[8 * k:8 * k + 8, :]
        y = yp_ref[8 * k:8 * k + 8, :]
        a = jnp.abs(x)
        b = jnp.abs(y)
        m = x != jnp.float32(0.85)
        a = jnp.where(m, a, jnp.float32(0.0))
        b = jnp.where(m, b, jnp.float32(0.0))
        p = a * b
        s = a + b
        for j in range(4):
            pi[j] = pi[j] + p[:, 128 * j:128 * j + 128]
            si[j] = si[j] + s[:, 128 * j:128 * j + 128]
    pcat = jnp.concatenate(pi, axis=1)
    scat = jnp.concatenate(si, axis=1)
    i = pl.program_id(0)

    @pl.when(i == 0)
    def _():
        oi_ref[...] = pcat
        os_ref[...] = scat

    @pl.when(i > 0)
    def _():
        oi_ref[...] += pcat
        os_ref[...] += scat


def _tc_partials(yt, yp):
    return pl.pallas_call(
        _tc_body,
        grid=(_G,),
        in_specs=[
            pl.BlockSpec((_BR, _COLS), lambda i: (i + _TC_OFF, 0)),
            pl.BlockSpec((_BR, _COLS), lambda i: (i + _TC_OFF, 0)),
        ],
        out_specs=[
            pl.BlockSpec((8, _COLS), lambda i: (0, 0)),
            pl.BlockSpec((8, _COLS), lambda i: (0, 0)),
        ],
        out_shape=[
            jax.ShapeDtypeStruct((8, _COLS), jnp.float32),
            jax.ShapeDtypeStruct((8, _COLS), jnp.float32),
        ],
        compiler_params=pltpu.CompilerParams(
            dimension_semantics=("arbitrary",),
        ),
    )(yt, yp)


@jax.jit
def _jaccard(y_true, y_pred):
    batch_size = y_true.shape[0]
    yt2 = y_true.reshape(_ROWS, _COLS)
    yp2 = y_pred.reshape(_ROWS, _COLS)
    sc_out = _sc_partials(y_true.reshape(_N), y_pred.reshape(_N))
    oi, os = _tc_partials(yt2, yp2)
    intersection = oi.sum() + sc_out[:, :_L].sum()
    sum_ = os.sum() + sc_out[:, _L:].sum()
    jac = (intersection + _SMOOTH) / (sum_ - intersection + _SMOOTH)
    return (1.0 - jac) * _SMOOTH / batch_size


def kernel(y_true, y_pred):
    return _jaccard(y_true, y_pred)


# speedup vs baseline: 1.6639x; 1.6639x over previous
import jax
import jax.numpy as jnp
from jax import lax
from jax.experimental import pallas as pl
from jax.experimental.pallas import tpu as pltpu
from jax.experimental.pallas import tpu_sc as plsc

_SMOOTH = 100.0
_N = 64 * 512 * 512
_COLS = 512
_ROWS = _N // _COLS

_NC = 2
_NS = 16
_NW = _NC * _NS
_CHR = 32
_SC_ROWS = 8192
_RW = _SC_ROWS // _NW
_CPW = _RW // _CHR
_L = 16

_BR = 2048
_TC_OFF = _SC_ROWS // _BR
_G = (_ROWS - _SC_ROWS) // _BR


def _sc_body(yt_hbm, yp_hbm, out_hbm, yt_buf, yp_buf, res_buf,
             sem_t0, sem_t1, sem_p0, sem_p1):
    wid = lax.axis_index("s") * _NC + lax.axis_index("c")
    base = wid * _RW
    sem_t = (sem_t0, sem_t1)
    sem_p = (sem_p0, sem_p1)

    def start(k, b):
        pltpu.async_copy(yt_hbm.at[pl.ds(base + k * _CHR, _CHR), :],
                         yt_buf.at[b], sem_t[b])
        pltpu.async_copy(yp_hbm.at[pl.ds(base + k * _CHR, _CHR), :],
                         yp_buf.at[b], sem_p[b])

    start(0, 0)
    start(1, 1)

    zeros = jnp.zeros((_L,), jnp.float32)
    init = (zeros, zeros, zeros, zeros)

    def outer(g, accs):
        for b in range(2):
            k = 2 * g + b
            pltpu.make_async_copy(yt_hbm.at[pl.ds(base, _CHR), :],
                                  yt_buf.at[b], sem_t[b]).wait()
            pltpu.make_async_copy(yp_hbm.at[pl.ds(base, _CHR), :],
                                  yp_buf.at[b], sem_p[b]).wait()

            def row_body(r, accs, b=b):
                accs = list(accs)
                for u in range(_COLS // _L):
                    yt = yt_buf[b, r, pl.ds(u * _L, _L)]
                    a = jnp.abs(yt)
                    p = jnp.abs(yp_buf[b, r, pl.ds(u * _L, _L)])
                    m = yt != jnp.float32(0.85)
                    a = jnp.where(m, a, jnp.float32(0.0))
                    p = jnp.where(m, p, jnp.float32(0.0))
                    j = u % 2
                    accs[j] = accs[j] + a * p
                    accs[2 + j] = accs[2 + j] + (a + p)
                return tuple(accs)

            accs = lax.fori_loop(0, _CHR, row_body, accs)

            @pl.when(k + 2 < _CPW)
            def _(k=k, b=b):
                start(k + 2, b)
        return accs

    accs = lax.fori_loop(0, _CPW // 2, outer, init)
    res_buf[pl.ds(0, _L)] = accs[0] + accs[1]
    res_buf[pl.ds(_L, _L)] = accs[2] + accs[3]
    pltpu.sync_copy(res_buf, out_hbm.at[wid])


def _sc_partials(yt2, yp2):
    return pl.kernel(
        _sc_body,
        out_type=jax.ShapeDtypeStruct((_NW, 2 * _L), jnp.float32),
        mesh=plsc.VectorSubcoreMesh(core_axis_name="c", subcore_axis_name="s"),
        scratch_types=[
            pltpu.VMEM((2, _CHR, _COLS), jnp.float32),
            pltpu.VMEM((2, _CHR, _COLS), jnp.float32),
            pltpu.VMEM((2 * _L,), jnp.float32),
            pltpu.SemaphoreType.DMA,
            pltpu.SemaphoreType.DMA,
            pltpu.SemaphoreType.DMA,
            pltpu.SemaphoreType.DMA,
        ],
    )(yt2, yp2)


def _tc_body(yt_ref, yp_ref, oi_ref, os_ref):
    pi = [jnp.zeros((8, 128), jnp.float32) for _ in range(4)]
    si = [jnp.zeros((8, 128), jnp.float32) for _ in range(4)]
    for k in range(_BR // 8):
        x = yt_ref[8 * k:8 * k + 8, :]
        y = yp_ref[8 * k:8 * k + 8, :]
        a = jnp.abs(x)
        b = jnp.abs(y)
        m = x != jnp.float32(0.85)
        a = jnp.where(m, a, jnp.float32(0.0))
        b = jnp.where(m, b, jnp.float32(0.0))
        p = a * b
        s = a + b
        for j in range(4):
            pi[j] = pi[j] + p[:, 128 * j:128 * j + 128]
            si[j] = si[j] + s[:, 128 * j:128 * j + 128]
    pcat = jnp.concatenate(pi, axis=1)
    scat = jnp.concatenate(si, axis=1)
    i = pl.program_id(0)

    @pl.when(i == 0)
    def _():
        oi_ref[...] = pcat
        os_ref[...] = scat

    @pl.when(i > 0)
    def _():
        oi_ref[...] += pcat
        os_ref[...] += scat


def _tc_partials(yt, yp):
    return pl.pallas_call(
        _tc_body,
        grid=(_G,),
        in_specs=[
            pl.BlockSpec((_BR, _COLS), lambda i: (i + _TC_OFF, 0)),
            pl.BlockSpec((_BR, _COLS), lambda i: (i + _TC_OFF, 0)),
        ],
        out_specs=[
            pl.BlockSpec((8, _COLS), lambda i: (0, 0)),
            pl.BlockSpec((8, _COLS), lambda i: (0, 0)),
        ],
        out_shape=[
            jax.ShapeDtypeStruct((8, _COLS), jnp.float32),
            jax.ShapeDtypeStruct((8, _COLS), jnp.float32),
        ],
        compiler_params=pltpu.CompilerParams(
            dimension_semantics=("arbitrary",),
        ),
    )(yt, yp)


@jax.jit
def _jaccard(y_true, y_pred):
    batch_size = y_true.shape[0]
    yt2 = y_true.reshape(_ROWS, _COLS)
    yp2 = y_pred.reshape(_ROWS, _COLS)
    sc_out = _sc_partials(yt2, yp2)
    oi, os = _tc_partials(yt2, yp2)
    intersection = oi.sum() + sc_out[:, :_L].sum()
    sum_ = os.sum() + sc_out[:, _L:].sum()
    jac = (intersection + _SMOOTH) / (sum_ - intersection + _SMOOTH)
    return (1.0 - jac) * _SMOOTH / batch_size


def kernel(y_true, y_pred):
    return _jaccard(y_true, y_pred)
